# baseline (speedup 1.0000x reference)
import functools

import jax
import jax.numpy as jnp
from jax import lax
from jax.experimental import pallas as pl
from jax.experimental.pallas import tpu as pltpu
from jax.experimental.pallas import tpu_sc as plsc

_MAX_LEN = 8192
_D = 1024
_B = 4
_NC = 2
_NS = 16
_NW = _NC * _NS
_ROWS = _MAX_LEN // _NW
_CHUNK = 32
_NCHUNK = _ROWS // _CHUNK

_mesh = plsc.VectorSubcoreMesh(core_axis_name="c", subcore_axis_name="s")


@functools.partial(
    pl.kernel,
    mesh=_mesh,
    out_type=jax.ShapeDtypeStruct((_B * _MAX_LEN, _D), jnp.float32),
    scratch_types=[pltpu.SemaphoreType.DMA],
)
def _bcast(pe_hbm, out_hbm, sem):
    wid = lax.axis_index("s") * _NC + lax.axis_index("c")
    base = wid * _ROWS
    cps = [
        pltpu.make_async_copy(
            pe_hbm.at[pl.ds(base, _ROWS)],
            out_hbm.at[pl.ds(b * _MAX_LEN + base, _ROWS)],
            sem,
        )
        for b in range(_B)
    ]
    for c in cps:
        c.start()
    for c in cps:
        c.wait()


def kernel(x, pe_table):
    del x
    out = _bcast(pe_table)
    return out.reshape(_B, _MAX_LEN, _D)


# speedup vs baseline: 55.6699x; 55.6699x over previous
import functools

import jax
import jax.numpy as jnp
from jax import lax
from jax.experimental import pallas as pl
from jax.experimental.pallas import tpu as pltpu
from jax.experimental.pallas import tpu_sc as plsc

_MAX_LEN = 8192
_D = 1024
_B = 4
_NC = 2
_NS = 16
_NW = _NC * _NS
_ROWS = _MAX_LEN // _NW
_CHUNK = 64
_NCHUNK = _ROWS // _CHUNK

_mesh = plsc.VectorSubcoreMesh(core_axis_name="c", subcore_axis_name="s")


@functools.partial(
    pl.kernel,
    mesh=_mesh,
    out_type=jax.ShapeDtypeStruct((_B * _MAX_LEN, _D), jnp.float32),
    scratch_types=[pltpu.VMEM((_CHUNK, _D), jnp.float32)],
)
def _bcast(pe_hbm, out_hbm, buf):
    wid = lax.axis_index("s") * _NC + lax.axis_index("c")
    base = wid * _ROWS
    for i in range(_NCHUNK):
        r0 = base + i * _CHUNK
        pltpu.sync_copy(pe_hbm.at[pl.ds(r0, _CHUNK)], buf)
        for b in range(_B):
            pltpu.sync_copy(buf, out_hbm.at[pl.ds(b * _MAX_LEN + r0, _CHUNK)])


def kernel(x, pe_table):
    del x
    out = _bcast(pe_table)
    return out.reshape(_B, _MAX_LEN, _D)
